# baseline (speedup 1.0000x reference)
import jax
import jax.numpy as jnp
import numpy as np
from jax import lax
from jax.experimental import pallas as pl
from jax.experimental.pallas import tpu as pltpu
from jax.experimental.pallas import tpu_sc as plsc

B = 1
N = 10000
E = 160000
D = 2
KS = 3
IN_DIM = 128
HID = 128
T = 2
S = 2 ** D
K = KS ** D

NC = 2
NS = 16
NW = NC * NS
CHUNK = 16
NIT = 320
HHID = HID // 2
EW = CHUNK * NIT
EPAD = EW * NW
ROWS_PER_SUB = 632
NACC = NS * ROWS_PER_SUB

_MESH = plsc.VectorSubcoreMesh(core_axis_name="c", subcore_axis_name="s")


def _mm_body(x_ref, w_ref, o_ref):
    o_ref[0] = jnp.dot(x_ref[...], w_ref[0],
                       preferred_element_type=jnp.float32)


def _xw_root(xin, wstack):
    kk, cin, _ = wstack.shape
    return pl.pallas_call(
        _mm_body,
        grid=(kk,),
        in_specs=[
            pl.BlockSpec((N, cin), lambda k: (0, 0)),
            pl.BlockSpec((1, cin, HID), lambda k: (k, 0, 0)),
        ],
        out_specs=pl.BlockSpec((1, N, HID), lambda k: (k, 0, 0)),
        out_shape=jax.ShapeDtypeStruct((kk, N, HID), jnp.float32),
    )(xin, wstack)


def _make_conv_body(G):
    def _conv_body(*refs):
        xws = refs[:G]
        gi0, gi1, gi2, gi3, b0h, b1h, b2h, b3h, dsth = refs[G:G + 9]
        outs = refs[G + 9:G + 9 + G]
        (idx0, idx1, idx2, idx3, bas0, bas1, bas2, bas3, dstv,
         r0, r1, r2, r3, msg, sem0, sem1, sem2, sem3, acc) = refs[G + 9 + G:]
        cid = lax.axis_index("c")
        sid = lax.axis_index("s")
        wid = sid * NC + cid
        base_row = sid * ROWS_PER_SUB
        ebase = wid * EW

        def _zero_own_rows():
            def _zrow(r, carry):
                for j in range(HID // 16):
                    msg[r, pl.ds(j * 16, 16)] = jnp.zeros((16,), jnp.float32)
                return carry
            lax.fori_loop(0, CHUNK, _zrow, 0)
            _nf = ROWS_PER_SUB // CHUNK
            for i in range(_nf):
                pltpu.sync_copy(msg,
                                acc.at[pl.ds(base_row + i * CHUNK, CHUNK)])
            pltpu.sync_copy(msg.at[pl.ds(0, ROWS_PER_SUB - _nf * CHUNK)],
                            acc.at[pl.ds(base_row + _nf * CHUNK,
                                         ROWS_PER_SUB - _nf * CHUNK)])

        _zero_own_rows()
        plsc.subcore_barrier()

        for g in range(G):
            xw = xws[g]

            def _iter(it, carry):
                off = ebase + it * CHUNK
                pltpu.sync_copy(gi0.at[pl.ds(off, CHUNK)], idx0)
                pltpu.sync_copy(gi1.at[pl.ds(off, CHUNK)], idx1)
                pltpu.sync_copy(gi2.at[pl.ds(off, CHUNK)], idx2)
                pltpu.sync_copy(gi3.at[pl.ds(off, CHUNK)], idx3)
                pltpu.sync_copy(b0h.at[pl.ds(off, CHUNK)], bas0)
                pltpu.sync_copy(b1h.at[pl.ds(off, CHUNK)], bas1)
                pltpu.sync_copy(b2h.at[pl.ds(off, CHUNK)], bas2)
                pltpu.sync_copy(b3h.at[pl.ds(off, CHUNK)], bas3)
                pltpu.sync_copy(dsth.at[pl.ds(off, CHUNK)], dstv.at[0])
                cp0 = pltpu.async_copy(xw.at[idx0], r0, sem0)
                cp1 = pltpu.async_copy(xw.at[idx1], r1, sem1)
                cp2 = pltpu.async_copy(xw.at[idx2], r2, sem2)
                cp3 = pltpu.async_copy(xw.at[idx3], r3, sem3)
                cp0.wait()
                cp1.wait()
                cp2.wait()
                cp3.wait()

                def _row(rr, c2):
                    w0 = bas0[rr, pl.ds(0, 16)]
                    w1 = bas1[rr, pl.ds(0, 16)]
                    w2 = bas2[rr, pl.ds(0, 16)]
                    w3 = bas3[rr, pl.ds(0, 16)]
                    for j in range(HID // 16):
                        slc = pl.ds(j * 16, 16)
                        msg[rr, slc] = (w0 * r0[rr, slc] + w1 * r1[rr, slc]
                                        + w2 * r2[rr, slc]
                                        + w3 * r3[rr, slc])
                    return c2
                lax.fori_loop(0, CHUNK, _row, 0)

                pltpu.sync_copy(msg, acc.at[dstv.at[0]], add=True)
                return carry
            lax.fori_loop(0, NIT, _iter, 0)

            plsc.subcore_barrier()
            pltpu.sync_copy(acc.at[pl.ds(base_row, ROWS_PER_SUB)],
                            outs[g].at[cid, pl.ds(base_row, ROWS_PER_SUB)])
            if g + 1 < G:
                _zero_own_rows()
                plsc.subcore_barrier()
    return _conv_body


def _sc_conv_group(xw_flats, gi, bas, dst_pad):
    G = len(xw_flats)
    conv = pl.kernel(
        _make_conv_body(G),
        out_type=[jax.ShapeDtypeStruct((NC, NACC, HID), jnp.float32)
                  for _ in range(G)],
        mesh=_MESH,
        scratch_types=[
            pltpu.VMEM((CHUNK,), jnp.int32),
            pltpu.VMEM((CHUNK,), jnp.int32),
            pltpu.VMEM((CHUNK,), jnp.int32),
            pltpu.VMEM((CHUNK,), jnp.int32),
            pltpu.VMEM((CHUNK, 16), jnp.float32),
            pltpu.VMEM((CHUNK, 16), jnp.float32),
            pltpu.VMEM((CHUNK, 16), jnp.float32),
            pltpu.VMEM((CHUNK, 16), jnp.float32),
            pltpu.VMEM((1, CHUNK), jnp.int32),
            pltpu.VMEM((CHUNK, HID), jnp.float32),
            pltpu.VMEM((CHUNK, HID), jnp.float32),
            pltpu.VMEM((CHUNK, HID), jnp.float32),
            pltpu.VMEM((CHUNK, HID), jnp.float32),
            pltpu.VMEM((CHUNK, HID), jnp.float32),
            pltpu.SemaphoreType.DMA,
            pltpu.SemaphoreType.DMA,
            pltpu.SemaphoreType.DMA,
            pltpu.SemaphoreType.DMA,
            pltpu.VMEM_SHARED((NACC, HID), jnp.float32),
        ],
    )
    return conv(*xw_flats, gi[0], gi[1], gi[2], gi[3],
                bas[0], bas[1], bas[2], bas[3], dst_pad)


def _cnt_body(dsth, out, dstv, ones_v, zer_v, acc):
    cid = lax.axis_index("c")
    sid = lax.axis_index("s")
    wid = sid * NC + cid
    base_row = sid * ROWS_PER_SUB

    def _fill(r, carry):
        for j in range(HID // 16):
            ones_v[r, pl.ds(j * 16, 16)] = jnp.ones((16,), jnp.float32)
            zer_v[r, pl.ds(j * 16, 16)] = jnp.zeros((16,), jnp.float32)
        return carry
    lax.fori_loop(0, CHUNK, _fill, 0)
    _nf = ROWS_PER_SUB // CHUNK
    for i in range(_nf):
        pltpu.sync_copy(zer_v, acc.at[pl.ds(base_row + i * CHUNK, CHUNK)])
    pltpu.sync_copy(zer_v.at[pl.ds(0, ROWS_PER_SUB - _nf * CHUNK)],
                    acc.at[pl.ds(base_row + _nf * CHUNK,
                                 ROWS_PER_SUB - _nf * CHUNK)])
    plsc.subcore_barrier()

    ebase = wid * EW

    def _iter(it, carry):
        off = ebase + it * CHUNK
        pltpu.sync_copy(dsth.at[pl.ds(off, CHUNK)], dstv.at[0])
        pltpu.sync_copy(ones_v, acc.at[dstv.at[0]], add=True)
        return carry
    lax.fori_loop(0, NIT, _iter, 0)

    plsc.subcore_barrier()
    pltpu.sync_copy(acc.at[pl.ds(base_row, ROWS_PER_SUB)],
                    out.at[cid, pl.ds(base_row, ROWS_PER_SUB)])


def _sc_cnt(dst_pad):
    cnt = pl.kernel(
        _cnt_body,
        out_type=jax.ShapeDtypeStruct((NC, NACC, HID), jnp.float32),
        mesh=_MESH,
        scratch_types=[
            pltpu.VMEM((1, CHUNK), jnp.int32),
            pltpu.VMEM((CHUNK, HID), jnp.float32),
            pltpu.VMEM((CHUNK, HID), jnp.float32),
            pltpu.VMEM_SHARED((NACC, HID), jnp.float32),
        ],
    )
    return cnt(dst_pad)


_RB = 2000


def _make_gates0_body(h):
    sl = slice(h * HHID, (h + 1) * HHID)

    def _gates0_body(c, ar, az, an, rr, rz, rn,
                     bxr, bxz, bxn, bhr, bhz, h1):
        inv = 1.0 / jnp.maximum(c[0][:, :1] + c[1][:, :1], 1.0)
        cxr = (ar[0] + ar[1])[:, sl] * inv + rr[...][:, sl] + bxr[...][:, sl]
        cxz = (az[0] + az[1])[:, sl] * inv + rz[...][:, sl] + bxz[...][:, sl]
        cxn = (an[0] + an[1])[:, sl] * inv + rn[...][:, sl] + bxn[...][:, sl]
        hrv = bhr[...][:, sl]
        r = jax.nn.sigmoid(cxr + hrv)
        z = jax.nn.sigmoid(cxz + bhz[...][:, sl])
        n = jnp.tanh(cxn + r * hrv)
        h1[...] = (1.0 - z) * n
    return _gates0_body


def _make_gates1_body(h):
    sl = slice(h * HHID, (h + 1) * HHID)

    def _gates1_body(c, ar, az, an, ah, aq, rr, rz, rn, rh, rq, hprev,
                     bxr, bxz, bxn, bhr, bhz, h2):
        inv = 1.0 / jnp.maximum(c[0][:, :1] + c[1][:, :1], 1.0)
        cxr = (ar[0] + ar[1])[:, sl] * inv + rr[...][:, sl] + bxr[...][:, sl]
        cxz = (az[0] + az[1])[:, sl] * inv + rz[...][:, sl] + bxz[...][:, sl]
        cxn = (an[0] + an[1])[:, sl] * inv + rn[...][:, sl] + bxn[...][:, sl]
        chr_ = (ah[0] + ah[1])[:, sl] * inv + rh[...][:, sl] + bhr[...][:, sl]
        chz = (aq[0] + aq[1])[:, sl] * inv + rq[...][:, sl] + bhz[...][:, sl]
        r = jax.nn.sigmoid(cxr + chr_)
        z = jax.nn.sigmoid(cxz + chz)
        n = jnp.tanh(cxn + r * chr_)
        h2[...] = (1.0 - z) * n + z * hprev[...][:, sl]
    return _gates1_body


def _agg_spec():
    return pl.BlockSpec((NC, _RB, HID), lambda i: (0, i, 0))


def _cnt_spec():
    return pl.BlockSpec((NC, _RB, HID), lambda i: (0, i, 0))


def _row_spec():
    return pl.BlockSpec((_RB, HID), lambda i: (i, 0))


def _bias_spec():
    return pl.BlockSpec((1, HID), lambda i: (0, 0))


def _half_out_spec():
    return pl.BlockSpec((_RB, HHID), lambda i: (i, 0))


def _gates0(cntp, aggs, roots, biases):
    halves = []
    for h in range(2):
        in_specs = ([_cnt_spec()] + [_agg_spec()] * 3
                    + [_row_spec()] * 3 + [_bias_spec()] * 5)
        halves.append(pl.pallas_call(
            _make_gates0_body(h),
            grid=(N // _RB,),
            in_specs=in_specs,
            out_specs=_half_out_spec(),
            out_shape=jax.ShapeDtypeStruct((N, HHID), jnp.float32),
        )(cntp, *[a[h] for a in aggs], *roots, *biases))
    return jnp.concatenate(halves, axis=1)


def _gates1(cntp, aggs, roots, hprev, biases):
    halves = []
    for h in range(2):
        in_specs = ([_cnt_spec()] + [_agg_spec()] * 5
                    + [_row_spec()] * 5 + [_row_spec()]
                    + [_bias_spec()] * 5)
        halves.append(pl.pallas_call(
            _make_gates1_body(h),
            grid=(N // _RB,),
            in_specs=in_specs,
            out_specs=_half_out_spec(),
            out_shape=jax.ShapeDtypeStruct((N, HHID), jnp.float32),
        )(cntp, *[a[h] for a in aggs], *roots, hprev, *biases))
    return jnp.concatenate(halves, axis=1)


def _conv_group(xin, gate_names, params, gi, bas, dst_pad):
    xw_flats, roots = [], []
    for g in gate_names:
        p = params[g]
        wstack = jnp.concatenate([p["weight"], p["root"][None]], axis=0)
        mm = _xw_root(xin, wstack)
        xw_flats.append(mm[:K].reshape(K * N, HID))
        roots.append(mm[K])
    aggs = _sc_conv_group(xw_flats, gi, bas, dst_pad)
    return [(a, a) for a in aggs], roots


def kernel(x, edge_index, edge_attr, params):
    src = edge_index[0]
    dst = edge_index[1]

    v = edge_attr * (KS - 1)
    bot = jnp.floor(v)
    frac = v - bot
    boti = bot.astype(jnp.int32)
    bits = (jnp.arange(S)[:, None] >> jnp.arange(D)[None, :]) & 1
    bitsf = bits.astype(jnp.float32)
    basis = jnp.prod(bitsf[None] * frac[:, None, :]
                     + (1.0 - bitsf[None]) * (1.0 - frac[:, None, :]),
                     axis=2)
    offsets = (KS ** jnp.arange(D)).astype(jnp.int32)
    wi = jnp.sum((boti[:, None, :] + bits[None]) * offsets[None, None, :],
                 axis=2)

    gidx = wi * N + src[:, None]
    pad = EPAD - E
    gi_all = jnp.concatenate(
        [gidx, jnp.zeros((pad, S), jnp.int32)], axis=0).T
    bas_all = jnp.concatenate(
        [basis, jnp.zeros((pad, S), jnp.float32)], axis=0).T
    dst_pad = jnp.concatenate(
        [dst, jnp.full((pad,), N, jnp.int32)], axis=0)
    gi = [gi_all[s] for s in range(S)]
    bas = [jnp.broadcast_to(bas_all[s][:, None], (EPAD, 16))
           for s in range(S)]

    cntp = _sc_cnt(dst_pad)

    biases = [params[g]["bias"].reshape(1, HID)
              for g in ("xr", "xz", "xn", "hr", "hz")]

    cur0 = x[0, :, :, 0] + 0.0 * cntp[0, :N, :1]
    aggs0, roots0 = _conv_group(cur0, ("xr", "xz", "xn"), params,
                                gi, bas, dst_pad)
    h1 = _gates0(cntp, aggs0, roots0, biases)

    cur1 = x[0, :, :, 1] + 0.0 * aggs0[2][1][0, :N, :1]
    aggs1x, roots1x = _conv_group(cur1, ("xr", "xz", "xn"), params,
                                  gi, bas, dst_pad)
    h1d = h1 + 0.0 * aggs1x[2][1][0, :N, :1]
    aggs1h, roots1h = _conv_group(h1d, ("hr", "hz"), params,
                                  gi, bas, dst_pad)
    h2 = _gates1(cntp, aggs1x + aggs1h, roots1x + roots1h, h1, biases)

    layer_output = jnp.stack([h1, h2], axis=-1).reshape(B, N, HID, T)
    last_h = h2.reshape(1, B, N, HID)
    return (layer_output, last_h)
